# Initial kernel scaffold; baseline (speedup 1.0000x reference)
import jax
import jax.numpy as jnp
from jax.experimental import pallas as pl


def kernel(input_words, input_docs, word_emb, doc_emb, W, b):
    raise NotImplementedError("write your pallas kernel here")


# speedup vs baseline: 1.2598x; 1.2598x over previous
import functools

import jax
import jax.numpy as jnp
from jax import lax
from jax.experimental import pallas as pl
from jax.experimental.pallas import tpu as pltpu
from jax.experimental.pallas import tpu_sc as plsc

_NC = 2
_NS = 16
_NW = _NC * _NS
_LANES = 16
_IDX_CHUNK = 128


def _gather_sum(words_flat, docs_flat, word_emb, doc_emb, B, C, D):
    b_per_w = B // _NW
    per_w_idx = b_per_w * C
    n_chunks = pl.cdiv(per_w_idx, _IDX_CHUNK)
    nd = D // _LANES

    widx = words_flat.reshape(_NW, n_chunks, _IDX_CHUNK)
    mesh = plsc.VectorSubcoreMesh(core_axis_name="c", subcore_axis_name="s")

    @functools.partial(
        pl.kernel,
        mesh=mesh,
        out_type=jax.ShapeDtypeStruct((B, D), jnp.float32),
        scratch_types=[
            pltpu.VMEM((n_chunks, _IDX_CHUNK), jnp.int32),
            pltpu.VMEM((b_per_w,), jnp.int32),
            pltpu.VMEM((per_w_idx, D), jnp.float32),
            pltpu.VMEM((b_per_w, D), jnp.float32),
            pltpu.VMEM((b_per_w, D), jnp.float32),
            pltpu.SemaphoreType.DMA,
        ],
    )
    def body(widx_hbm, didx_hbm, wtab_hbm, dtab_hbm, out_hbm,
             widx_v, didx_v, wrows_v, drows_v, out_v, sem):
        wid = lax.axis_index("s") * _NC + lax.axis_index("c")
        base = wid * b_per_w
        pltpu.sync_copy(widx_hbm.at[wid], widx_v)
        pltpu.sync_copy(didx_hbm.at[pl.ds(base, b_per_w)], didx_v)
        copies = []
        for j in range(n_chunks):
            copies.append(pltpu.async_copy(
                wtab_hbm.at[widx_v.at[j]],
                wrows_v.at[pl.ds(j * _IDX_CHUNK, _IDX_CHUNK)], sem))
        copies.append(pltpu.async_copy(dtab_hbm.at[didx_v], drows_v, sem))
        for cp in copies:
            cp.wait()

        def accum(lb, carry):
            accs = [drows_v[lb, pl.ds(d * _LANES, _LANES)] for d in range(nd)]
            for j in range(C):
                row = lb * C + j
                for d in range(nd):
                    accs[d] = accs[d] + wrows_v[row, pl.ds(d * _LANES, _LANES)]
            for d in range(nd):
                out_v[lb, pl.ds(d * _LANES, _LANES)] = accs[d]
            return carry

        lax.fori_loop(0, b_per_w, accum, 0)
        pltpu.sync_copy(out_v, out_hbm.at[pl.ds(base, b_per_w)])

    return body(widx, docs_flat, word_emb, doc_emb)


def _projection(hidden, W, bias, VB=2048):
    B, D = hidden.shape
    V = W.shape[0]
    nv = pl.cdiv(V, VB)
    bias2 = bias.reshape(1, V)

    def mm(h_ref, w_ref, b_ref, o_ref):
        o_ref[...] = lax.dot_general(
            h_ref[...], w_ref[...],
            dimension_numbers=(((1,), (1,)), ((), ())),
            preferred_element_type=jnp.float32,
        ) + b_ref[...]

    return pl.pallas_call(
        mm,
        grid=(nv,),
        in_specs=[
            pl.BlockSpec((B, D), lambda i: (0, 0)),
            pl.BlockSpec((VB, D), lambda i: (i, 0)),
            pl.BlockSpec((1, VB), lambda i: (0, i)),
        ],
        out_specs=pl.BlockSpec((B, VB), lambda i: (0, i)),
        out_shape=jax.ShapeDtypeStruct((B, V), jnp.float32),
    )(hidden, W, bias2)


def kernel(input_words, input_docs, word_emb, doc_emb, W, b):
    C, B = input_words.shape
    V, D = W.shape
    words_flat = input_words.T.reshape(B * C).astype(jnp.int32)
    docs_flat = input_docs.reshape(B).astype(jnp.int32)
    hidden = _gather_sum(words_flat, docs_flat, word_emb, doc_emb, B, C, D)
    out = _projection(hidden, W, b)
    return out[None, :, :]
